# baseline (speedup 1.0000x reference)
import functools

import jax
import jax.numpy as jnp
from jax import lax
from jax.experimental import pallas as pl
from jax.experimental.pallas import tpu as pltpu

_BIG = 1e30
_ROWS = 64


def _softplus(x):
    return jnp.maximum(x, 0.0) + jnp.log1p(jnp.exp(-jnp.abs(x)))


def _shift_right(x, d, fill):
    b = x.shape[0]
    pad = jnp.full((b, d), fill, dtype=x.dtype)
    return jnp.concatenate([pad, x[:, :-d]], axis=1)


def _cummin(x, lo=0):
    n = x.shape[-1]
    parts = ([x] if lo == 0 else []) + [
        _shift_right(x, k, _BIG) for k in range(max(lo, 1), 9 - lo)
    ]
    x = functools.reduce(jnp.minimum, parts)
    d = 8
    while d < n:
        parts = [x] + [_shift_right(x, d * k, _BIG) for k in range(1, 8) if d * k < n]
        x = functools.reduce(jnp.minimum, parts)
        d *= 8
    return x


def _excl_prefix_sum(wr, tri):
    return lax.dot_general(
        wr, tri, (((1,), (0,)), ((), ())),
        precision=lax.Precision.HIGHEST,
        preferred_element_type=jnp.float32,
    )


def _row_update(prev_im, cur_im, prev_d, tri):
    cur_l = jnp.concatenate([cur_im[:, 1:], cur_im[:, -1:]], axis=1)
    wd = _softplus((prev_im + cur_im) * 0.5)
    wdgl = _softplus((prev_im + cur_l) * 0.5)
    wr = _softplus((cur_im + cur_l) * 0.5)
    p = _excl_prefix_sum(wr, tri)
    m1 = _cummin(prev_d + (wd - p))
    m2 = _cummin(prev_d + (wdgl - (p + wr)), lo=1)
    return p + jnp.minimum(m1, m2)


def _first_row(cur, tri):
    right = jnp.concatenate([cur[:, 1:], cur[:, -1:]], axis=1)
    wr = _softplus((cur + right) * 0.5)
    return _excl_prefix_sum(wr, tri)


def _dp_body(tri_ref, img_ref, out_ref, prev_img, carry):
    g = pl.program_id(0)
    cur = img_ref[...]
    tri = tri_ref[...]
    rows = [cur[r] for r in range(_ROWS)]

    @pl.when(g == 0)
    def _init():
        d = _first_row(rows[0], tri)
        for r in range(1, _ROWS):
            d = _row_update(rows[r - 1], rows[r], d, tri)
        carry[...] = d
        prev_img[...] = rows[_ROWS - 1]

    @pl.when(g > 0)
    def _step():
        d = carry[...]
        pim = prev_img[...]
        for r in range(_ROWS):
            d = _row_update(pim, rows[r], d, tri)
            pim = rows[r]
        carry[...] = d
        prev_img[...] = pim

    @pl.when(g == pl.num_programs(0) - 1)
    def _emit():
        out_ref[...] = carry[...]


@jax.jit
def kernel(images):
    b, h, w = images.shape
    imgs_t = images.transpose(1, 0, 2)
    tri = jnp.triu(jnp.ones((w, w), jnp.float32), k=1)
    out = pl.pallas_call(
        _dp_body,
        grid=(h // _ROWS,),
        in_specs=[
            pl.BlockSpec((w, w), lambda g: (0, 0)),
            pl.BlockSpec((_ROWS, b, w), lambda g: (g, 0, 0)),
        ],
        out_specs=pl.BlockSpec((b, w), lambda g: (0, 0)),
        out_shape=jax.ShapeDtypeStruct((b, w), jnp.float32),
        scratch_shapes=[
            pltpu.VMEM((b, w), jnp.float32),
            pltpu.VMEM((b, w), jnp.float32),
        ],
    )(tri, imgs_t)
    return out[:, -1]


# speedup vs baseline: 1.8452x; 1.8452x over previous
import functools

import jax
import jax.numpy as jnp
from jax import lax
from jax.experimental import pallas as pl
from jax.experimental.pallas import tpu as pltpu

_BIG = 1e30
_ROWS = 32


def _softplus(x):
    return jnp.maximum(x, 0.0) + jnp.log1p(jnp.exp(-jnp.abs(x)))


def _shift_right(x, d, fill):
    b = x.shape[0]
    pad = jnp.full((b, d), fill, dtype=x.dtype)
    return jnp.concatenate([pad, x[:, :-d]], axis=1)


def _cummin(x, lo=0):
    n = x.shape[-1]
    parts = ([x] if lo == 0 else []) + [
        _shift_right(x, k, _BIG) for k in range(max(lo, 1), 9 - lo)
    ]
    x = functools.reduce(jnp.minimum, parts)
    d = 8
    while d < n:
        parts = [x] + [_shift_right(x, d * k, _BIG) for k in range(1, 8) if d * k < n]
        x = functools.reduce(jnp.minimum, parts)
        d *= 8
    return x


def _excl_prefix_sum(wr, tri):
    return lax.dot_general(
        wr, tri, (((1,), (0,)), ((), ())),
        precision=lax.Precision.HIGHEST,
        preferred_element_type=jnp.float32,
    )


def _row_update(prev_im, cur_im, prev_d, tri):
    cur_l = jnp.concatenate([cur_im[:, 1:], cur_im[:, -1:]], axis=1)
    wd = _softplus((prev_im + cur_im) * 0.5)
    wdgl = _softplus((prev_im + cur_l) * 0.5)
    wr = _softplus((cur_im + cur_l) * 0.5)
    p = _excl_prefix_sum(wr, tri)
    m1 = _cummin(prev_d + (wd - p))
    m2 = _cummin(prev_d + (wdgl - (p + wr)), lo=1)
    return p + jnp.minimum(m1, m2)


def _first_row(cur, tri):
    right = jnp.concatenate([cur[:, 1:], cur[:, -1:]], axis=1)
    wr = _softplus((cur + right) * 0.5)
    return _excl_prefix_sum(wr, tri)


def _dp_body(tri_ref, img_ref, out_ref, prev_img, carry):
    g = pl.program_id(0)
    cur = img_ref[...]
    tri = tri_ref[...]
    rows = [cur[r] for r in range(_ROWS)]

    @pl.when(g == 0)
    def _init():
        d = _first_row(rows[0], tri)
        for r in range(1, _ROWS):
            d = _row_update(rows[r - 1], rows[r], d, tri)
        carry[...] = d
        prev_img[...] = rows[_ROWS - 1]

    @pl.when(g > 0)
    def _step():
        d = carry[...]
        pim = prev_img[...]
        for r in range(_ROWS):
            d = _row_update(pim, rows[r], d, tri)
            pim = rows[r]
        carry[...] = d
        prev_img[...] = pim

    @pl.when(g == pl.num_programs(0) - 1)
    def _emit():
        out_ref[...] = carry[...]


@jax.jit
def kernel(images):
    b, h, w = images.shape
    imgs_t = images.transpose(1, 0, 2)
    tri = jnp.triu(jnp.ones((w, w), jnp.float32), k=1)
    out = pl.pallas_call(
        _dp_body,
        grid=(h // _ROWS,),
        in_specs=[
            pl.BlockSpec((w, w), lambda g: (0, 0)),
            pl.BlockSpec((_ROWS, b, w), lambda g: (g, 0, 0)),
        ],
        out_specs=pl.BlockSpec((b, w), lambda g: (0, 0)),
        out_shape=jax.ShapeDtypeStruct((b, w), jnp.float32),
        scratch_shapes=[
            pltpu.VMEM((b, w), jnp.float32),
            pltpu.VMEM((b, w), jnp.float32),
        ],
    )(tri, imgs_t)
    return out[:, -1]
